# baseline (speedup 1.0000x reference)
import jax
import jax.numpy as jnp
from jax.experimental import pallas as pl

_R = 16384
_TT = 200
_C = 24
_TB = 10


def _score_kernel(idx_ref, yp_ref, b_ref, out_ref):
    step = pl.program_id(0)

    b = b_ref[...]
    iota = jax.lax.broadcasted_iota(jnp.int32, (_C, _R), 0)

    partial = jnp.zeros((), jnp.float32)
    for u in range(_TB):
        idx = idx_ref[u].reshape(1, _R)
        yp = yp_ref[u * _C:(u + 1) * _C, :]
        onehot = (idx == iota).astype(jnp.float32)
        s = jax.lax.dot_general(yp, onehot, (((1,), (1,)), ((), ())),
                                preferred_element_type=jnp.float32)
        partial = partial + jnp.sum(s * b)

    @pl.when(step == 0)
    def _():
        out_ref[...] = jnp.zeros_like(out_ref)

    out_ref[...] = out_ref[...] + partial


def kernel(y_true, y_pred, B):
    ypt = y_pred.transpose(1, 2, 0).reshape(_TT * _C, _R)
    idx = y_true.T.reshape(_TT, 1, _R).astype(jnp.int32)

    out = pl.pallas_call(
        _score_kernel,
        grid=(_TT // _TB,),
        in_specs=[
            pl.BlockSpec((_TB, 1, _R), lambda i: (i, 0, 0)),
            pl.BlockSpec((_TB * _C, _R), lambda i: (i, 0)),
            pl.BlockSpec((_C, _C), lambda i: (0, 0)),
        ],
        out_specs=pl.BlockSpec((1, 1), lambda i: (0, 0)),
        out_shape=jax.ShapeDtypeStruct((1, 1), jnp.float32),
    )(idx, ypt, B)
    return out[0, 0]
